# baseline (speedup 1.0000x reference)
import functools

import jax
import jax.numpy as jnp
from jax import lax
from jax.experimental import pallas as pl
from jax.experimental.pallas import tpu as pltpu
from jax.experimental.pallas import tpu_sc as plsc

INF2 = 1e16
BACKGROUND_LABEL = 10
ATTR_BACKGROUND_LABEL = 9
RADIUS = 1.5
ALPHA = 2.5

L = 16
NB = 4
NC, NS = 2, 16
NW = NC * NS

_DNUMS = lax.GatherDimensionNumbers(
    offset_dims=(), collapsed_slice_dims=(0,), start_index_map=(0,))


def _lane(v, k):
    idx = jnp.full((L, 1), k, jnp.int32)
    return lax.gather(v, idx, _DNUMS, (1,),
                      mode=lax.GatherScatterMode.PROMISE_IN_BOUNDS)


def _atan2(y, x):
    ay, ax = jnp.abs(y), jnp.abs(x)
    mx = jnp.maximum(ax, ay)
    mn = jnp.minimum(ax, ay)
    a = mn / jnp.maximum(mx, 1e-30)
    s = a * a
    p = jnp.float32(-0.01172120)
    for c in (0.05265332, -0.11643287, 0.19354346, -0.33262347, 0.99997726):
        p = p * s + jnp.float32(c)
    r = p * a
    r = jnp.where(ay > ax, jnp.float32(jnp.pi / 2) - r, r)
    r = jnp.where(x < 0, jnp.float32(jnp.pi) - r, r)
    return jnp.where(y < 0, -r, r)


def _sqrt(x):
    x = jnp.maximum(x, 1e-24)
    i = lax.bitcast_convert_type(x, jnp.int32)
    y = lax.bitcast_convert_type((i >> 1) + jnp.int32(0x1FBD1DF5), jnp.float32)
    for _ in range(3):
        y = 0.5 * (y + x / y)
    return y


def _body(in_hbm, tab_hbm, tft_hbm,
          bt_hbm, bt3_hbm, cen_hbm, out0_hbm, out1_hbm, out2_hbm,
          inv, tf, tft, cand, btv, bt3v, cenv, o0v, o1v, o2v,
          sem, chunk):
    wid = lax.axis_index("s") * NC + lax.axis_index("c")
    base = wid * chunk
    copies = [
        pltpu.async_copy(in_hbm.at[:, pl.ds(base, chunk)], inv, sem),
        pltpu.async_copy(tab_hbm, tf, sem),
        pltpu.async_copy(tft_hbm, tft, sem),
    ]
    for c in copies:
        c.wait()

    for j in range(64 // L):
        y_sl = pl.ds(10 * 64 + j * L, L)
        a_sl = pl.ds(13 * 64 + j * L, L)
        b_sl = pl.ds(14 * 64 + j * L, L)
        tf[y_sl] = tf[y_sl] - _atan2(tf[a_sl], tf[b_sl])

    def block(i, _):
        iota16 = lax.iota(jnp.int32, L)
        pts = []
        for b in range(NB):
            sl = pl.ds(i * (NB * L) + b * L, L)
            pts.append((inv[0, sl], inv[1, sl], inv[2, sl], inv[3, sl],
                        inv[4, sl] * RADIUS))

        xmn = jnp.min(jnp.minimum(jnp.minimum(pts[0][0], pts[1][0]),
                                  jnp.minimum(pts[2][0], pts[3][0])))
        xmx = jnp.max(jnp.maximum(jnp.maximum(pts[0][0], pts[1][0]),
                                  jnp.maximum(pts[2][0], pts[3][0])))
        ymn = jnp.min(jnp.minimum(jnp.minimum(pts[0][1], pts[1][1]),
                                  jnp.minimum(pts[2][1], pts[3][1])))
        ymx = jnp.max(jnp.maximum(jnp.maximum(pts[0][1], pts[1][1]),
                                  jnp.maximum(pts[2][1], pts[3][1])))
        srm = jnp.max(jnp.maximum(jnp.maximum(pts[0][4], pts[1][4]),
                                  jnp.maximum(pts[2][4], pts[3][4])))
        for k in range(5):
            cand[pl.ds(k * L, L)] = jnp.full((L,), 64, jnp.int32)
        off = jnp.int32(0)
        for j in range(4):
            gidx = iota16 + j * L
            cxs = plsc.load_gather(tft, [gidx * L])
            cys = plsc.load_gather(tft, [gidx * L + 1])
            m = ((jnp.abs(cxs - jnp.clip(cxs, xmn, xmx)) < srm) &
                 (jnp.abs(cys - jnp.clip(cys, ymn, ymx)) < srm))
            plsc.store_compressed(cand.at[pl.ds(off, L)], gidx, mask=m)
            off = off + plsc.all_reduce_population_count(m)[0]
        nch = lax.shift_right_logical(off + 7, 3)

        def chunk_step(k, carry):
            candv = cand[pl.ds(k * 8, L)]
            out = list(carry)
            for j in range(8):
                gv = _lane(candv, j)
                rowv = plsc.load_gather(tft, [gv * L + iota16])
                cx = _lane(rowv, 0)
                cy = _lane(rowv, 1)
                b0 = _lane(rowv, 2)
                b1 = _lane(rowv, 3)
                b2 = _lane(rowv, 4)
                b3 = _lane(rowv, 5)
                for b in range(NB):
                    best, bidx = out[2 * b], out[2 * b + 1]
                    xs, ys, rlo, rhi, sr = pts[b]
                    dx = xs - cx
                    dy = ys - cy
                    d2 = dx * dx + dy * dy
                    ing = jnp.maximum(jnp.abs(dx), jnp.abs(dy)) < sr
                    mrd = jnp.maximum(jnp.maximum(xs - b0, ys - b1),
                                      jnp.maximum(b2 - xs, b3 - ys))
                    ok = ing & (mrd >= rlo) & (mrd <= rhi)
                    ltm = ok & (d2 < best)
                    out[2 * b] = jnp.where(ltm, d2, best)
                    out[2 * b + 1] = jnp.where(ltm, gv, bidx)
            return tuple(out)

        init = []
        for b in range(NB):
            init.append(jnp.full((L,), INF2, jnp.float32))
            init.append(jnp.zeros((L,), jnp.int32))
        res = lax.fori_loop(0, nch, chunk_step, tuple(init))

        for b in range(NB):
            best, bidx = res[2 * b], res[2 * b + 1]
            xs, ys, rlo, rhi, sr = pts[b]
            sl = pl.ds(i * (NB * L) + b * L, L)

            def gat(row, bidx=bidx):
                return plsc.load_gather(
                    tf, [jnp.full((L,), row * 64, jnp.int32) + bidx])

            cxg = gat(0)
            cyg = gat(1)
            dxg = xs - cxg
            dyg = ys - cyg
            dg = _sqrt(dxg * dxg + dyg * dyg)
            cent = jnp.exp((-ALPHA) * (dg / (jnp.float32(1.414) * sr)))

            is_bg = best == jnp.float32(INF2)
            labg = plsc.bitcast(gat(15), jnp.int32)
            lab3g = plsc.bitcast(gat(16), jnp.int32)
            attg = plsc.bitcast(gat(17), jnp.int32)
            o0v[sl] = jnp.where(is_bg, BACKGROUND_LABEL, labg)
            o1v[sl] = jnp.where(is_bg, BACKGROUND_LABEL, lab3g)
            o2v[sl] = jnp.where(is_bg, ATTR_BACKGROUND_LABEL, attg)

            off = i * (NB * L) + b * L
            cenv[sl] = cent
            for c, v in enumerate([xs - gat(2), ys - gat(3),
                                   gat(4) - xs, gat(5) - ys]):
                btv[pl.ds(c * chunk + off, L)] = v
            for c, v in enumerate([dxg, dyg, gat(6), gat(7), gat(8), gat(9),
                                   gat(10), gat(11), gat(12)]):
                bt3v[pl.ds(c * chunk + off, L)] = v
        return 0

    lax.fori_loop(0, chunk // (NB * L), block, 0)
    Ppad = chunk * NW
    outs = [
        pltpu.async_copy(cenv, cen_hbm.at[pl.ds(base, chunk)], sem),
        pltpu.async_copy(o0v, out0_hbm.at[pl.ds(base, chunk)], sem),
        pltpu.async_copy(o1v, out1_hbm.at[pl.ds(base, chunk)], sem),
        pltpu.async_copy(o2v, out2_hbm.at[pl.ds(base, chunk)], sem),
    ]
    for c in range(4):
        outs.append(pltpu.async_copy(
            btv.at[pl.ds(c * chunk, chunk)],
            bt_hbm.at[pl.ds(c * Ppad + base, chunk)], sem))
    for c in range(9):
        outs.append(pltpu.async_copy(
            bt3v.at[pl.ds(c * chunk, chunk)],
            bt3_hbm.at[pl.ds(c * Ppad + base, chunk)], sem))
    for c in outs:
        c.wait()


def kernel(gt_bboxes, gt_labels, gt_bboxes_3d, gt_labels_3d, centers2d,
           depths, attr_labels, points, regress_ranges, stride_pt):
    P = points.shape[0]
    blk = NB * L
    chunk = ((P + NW - 1) // NW + blk - 1) // blk * blk
    Ppad = chunk * NW

    pad = Ppad - P
    in2d = jnp.concatenate([
        jnp.stack([points[:, 0], points[:, 1], regress_ranges[:, 0],
                   regress_ranges[:, 1], stride_pt]),
        jnp.ones((5, pad), jnp.float32),
    ], axis=1)

    tab = jnp.concatenate([
        centers2d.T,
        gt_bboxes.T,
        depths[None, :],
        gt_bboxes_3d[:, 3:9].T,
        gt_bboxes_3d[:, 0][None, :],
        gt_bboxes_3d[:, 2][None, :],
        lax.bitcast_convert_type(
            jnp.stack([gt_labels, gt_labels_3d, attr_labels]).astype(jnp.int32),
            jnp.float32),
    ]).astype(jnp.float32).reshape(-1)
    tft = jnp.pad(
        jnp.concatenate([centers2d, gt_bboxes], axis=1),
        ((0, 2), (0, L - 6)), constant_values=1e12).astype(jnp.float32)
    tft = tft.at[:64, 6:].set(0.0).at[65].set(0.0).reshape(-1)

    mesh = plsc.VectorSubcoreMesh(core_axis_name="c", subcore_axis_name="s")
    kfn = functools.partial(
        pl.kernel,
        out_type=[jax.ShapeDtypeStruct((4 * Ppad,), jnp.float32),
                  jax.ShapeDtypeStruct((9 * Ppad,), jnp.float32),
                  jax.ShapeDtypeStruct((Ppad,), jnp.float32),
                  jax.ShapeDtypeStruct((Ppad,), jnp.int32),
                  jax.ShapeDtypeStruct((Ppad,), jnp.int32),
                  jax.ShapeDtypeStruct((Ppad,), jnp.int32)],
        mesh=mesh,
        compiler_params=pltpu.CompilerParams(needs_layout_passes=False),
        scratch_types=[pltpu.VMEM((5, chunk), jnp.float32),
                       pltpu.VMEM((18 * 64,), jnp.float32),
                       pltpu.VMEM((66 * L,), jnp.float32),
                       pltpu.VMEM((80,), jnp.int32),
                       pltpu.VMEM((chunk * 4,), jnp.float32),
                       pltpu.VMEM((chunk * 9,), jnp.float32),
                       pltpu.VMEM((chunk,), jnp.float32),
                       pltpu.VMEM((chunk,), jnp.int32),
                       pltpu.VMEM((chunk,), jnp.int32),
                       pltpu.VMEM((chunk,), jnp.int32),
                       pltpu.SemaphoreType.DMA],
    )(functools.partial(_body, chunk=chunk))
    btf, bt3f, cenf, o0, o1, o2 = kfn(in2d, tab, tft)

    labels = o0[:P]
    labels_3d = o1[:P]
    attrs = o2[:P]
    b2 = btf.reshape(4, Ppad)
    b3 = bt3f.reshape(9, Ppad)
    bt = jnp.stack([b2[c, :P] for c in range(4)], axis=-1)
    bt3d = jnp.stack([b3[c, :P] for c in range(9)], axis=-1)
    cent = cenf[:P]
    return (labels, bt, labels_3d, bt3d, cent, attrs)


# speedup vs baseline: 1.5816x; 1.5816x over previous
import functools

import jax
import jax.numpy as jnp
from jax import lax
from jax.experimental import pallas as pl
from jax.experimental.pallas import tpu as pltpu
from jax.experimental.pallas import tpu_sc as plsc

INF2 = 1e16
BACKGROUND_LABEL = 10
ATTR_BACKGROUND_LABEL = 9
RADIUS = 1.5
ALPHA = 2.5

L = 16
NB = 4
NC, NS = 2, 16
NW = NC * NS

_DNUMS = lax.GatherDimensionNumbers(
    offset_dims=(), collapsed_slice_dims=(0,), start_index_map=(0,))


def _lane(v, k):
    idx = jnp.full((L, 1), k, jnp.int32)
    return lax.gather(v, idx, _DNUMS, (1,),
                      mode=lax.GatherScatterMode.PROMISE_IN_BOUNDS)


def _atan2(y, x):
    ay, ax = jnp.abs(y), jnp.abs(x)
    mx = jnp.maximum(ax, ay)
    mn = jnp.minimum(ax, ay)
    a = mn / jnp.maximum(mx, 1e-30)
    s = a * a
    p = jnp.float32(-0.01172120)
    for c in (0.05265332, -0.11643287, 0.19354346, -0.33262347, 0.99997726):
        p = p * s + jnp.float32(c)
    r = p * a
    r = jnp.where(ay > ax, jnp.float32(jnp.pi / 2) - r, r)
    r = jnp.where(x < 0, jnp.float32(jnp.pi) - r, r)
    return jnp.where(y < 0, -r, r)


def _sqrt(x):
    x = jnp.maximum(x, 1e-24)
    i = lax.bitcast_convert_type(x, jnp.int32)
    y = lax.bitcast_convert_type((i >> 1) + jnp.int32(0x1FBD1DF5), jnp.float32)
    for _ in range(3):
        y = 0.5 * (y + x / y)
    return y


def _body(in_hbm, tab_hbm, tft_hbm,
          bt_hbm, bt3_hbm, cen_hbm, out0_hbm, out1_hbm, out2_hbm,
          inv, tf, tft, cand, btv, bt3v, cenv, o0v, o1v, o2v,
          sem, chunk):
    wid = lax.axis_index("s") * NC + lax.axis_index("c")
    base = wid * chunk
    copies = [
        pltpu.async_copy(in_hbm.at[:, pl.ds(base, chunk)], inv, sem),
        pltpu.async_copy(tab_hbm, tf, sem),
        pltpu.async_copy(tft_hbm, tft, sem),
    ]
    for c in copies:
        c.wait()

    for j in range(64 // L):
        y_sl = pl.ds(10 * 64 + j * L, L)
        a_sl = pl.ds(13 * 64 + j * L, L)
        b_sl = pl.ds(14 * 64 + j * L, L)
        tf[y_sl] = tf[y_sl] - _atan2(tf[a_sl], tf[b_sl])

    def block(i, _):
        iota16 = lax.iota(jnp.int32, L)
        pts = []
        for b in range(NB):
            sl = pl.ds(i * (NB * L) + b * L, L)
            pts.append((inv[0, sl], inv[1, sl], inv[2, sl], inv[3, sl],
                        inv[4, sl] * RADIUS))

        xmn = jnp.min(jnp.minimum(jnp.minimum(pts[0][0], pts[1][0]),
                                  jnp.minimum(pts[2][0], pts[3][0])))
        xmx = jnp.max(jnp.maximum(jnp.maximum(pts[0][0], pts[1][0]),
                                  jnp.maximum(pts[2][0], pts[3][0])))
        ymn = jnp.min(jnp.minimum(jnp.minimum(pts[0][1], pts[1][1]),
                                  jnp.minimum(pts[2][1], pts[3][1])))
        ymx = jnp.max(jnp.maximum(jnp.maximum(pts[0][1], pts[1][1]),
                                  jnp.maximum(pts[2][1], pts[3][1])))
        srm = jnp.max(jnp.maximum(jnp.maximum(pts[0][4], pts[1][4]),
                                  jnp.maximum(pts[2][4], pts[3][4])))
        for k in range(5):
            cand[pl.ds(k * L, L)] = jnp.full((L,), 64, jnp.int32)
        off = jnp.int32(0)
        for j in range(4):
            gidx = iota16 + j * L
            cxs = plsc.load_gather(tft, [gidx * L])
            cys = plsc.load_gather(tft, [gidx * L + 1])
            m = ((jnp.abs(cxs - jnp.clip(cxs, xmn, xmx)) < srm) &
                 (jnp.abs(cys - jnp.clip(cys, ymn, ymx)) < srm))
            plsc.store_compressed(cand.at[pl.ds(off, L)], gidx, mask=m)
            off = off + plsc.all_reduce_population_count(m)[0]
        nch = lax.shift_right_logical(off + 15, 4)

        def chunk_step(k, carry):
            candv = cand[pl.ds(k * L, L)]
            out = list(carry)
            for j in range(L):
                gv = _lane(candv, j)
                rowv = plsc.load_gather(tft, [gv * L + iota16])
                cx = _lane(rowv, 0)
                cy = _lane(rowv, 1)
                b0 = _lane(rowv, 2)
                b1 = _lane(rowv, 3)
                b2 = _lane(rowv, 4)
                b3 = _lane(rowv, 5)
                for b in range(NB):
                    best, bidx = out[2 * b], out[2 * b + 1]
                    xs, ys, rlo, rhi, sr = pts[b]
                    dx = xs - cx
                    dy = ys - cy
                    d2 = dx * dx + dy * dy
                    ing = jnp.maximum(jnp.abs(dx), jnp.abs(dy)) < sr
                    mrd = jnp.maximum(jnp.maximum(xs - b0, ys - b1),
                                      jnp.maximum(b2 - xs, b3 - ys))
                    ok = ing & (mrd >= rlo) & (mrd <= rhi)
                    ltm = ok & (d2 < best)
                    out[2 * b] = jnp.where(ltm, d2, best)
                    out[2 * b + 1] = jnp.where(ltm, gv, bidx)
            return tuple(out)

        init = []
        for b in range(NB):
            init.append(jnp.full((L,), INF2, jnp.float32))
            init.append(jnp.zeros((L,), jnp.int32))
        res = lax.fori_loop(0, nch, chunk_step, tuple(init))

        for b in range(NB):
            best, bidx = res[2 * b], res[2 * b + 1]
            xs, ys, rlo, rhi, sr = pts[b]
            sl = pl.ds(i * (NB * L) + b * L, L)

            def gat(row, bidx=bidx):
                return plsc.load_gather(
                    tf, [jnp.full((L,), row * 64, jnp.int32) + bidx])

            cxg = gat(0)
            cyg = gat(1)
            dxg = xs - cxg
            dyg = ys - cyg
            dg = _sqrt(dxg * dxg + dyg * dyg)
            cent = jnp.exp((-ALPHA) * (dg / (jnp.float32(1.414) * sr)))

            is_bg = best == jnp.float32(INF2)
            labg = plsc.bitcast(gat(15), jnp.int32)
            lab3g = plsc.bitcast(gat(16), jnp.int32)
            attg = plsc.bitcast(gat(17), jnp.int32)
            o0v[sl] = jnp.where(is_bg, BACKGROUND_LABEL, labg)
            o1v[sl] = jnp.where(is_bg, BACKGROUND_LABEL, lab3g)
            o2v[sl] = jnp.where(is_bg, ATTR_BACKGROUND_LABEL, attg)

            off = i * (NB * L) + b * L
            cenv[sl] = cent
            for c, v in enumerate([xs - gat(2), ys - gat(3),
                                   gat(4) - xs, gat(5) - ys]):
                btv[pl.ds(c * chunk + off, L)] = v
            for c, v in enumerate([dxg, dyg, gat(6), gat(7), gat(8), gat(9),
                                   gat(10), gat(11), gat(12)]):
                bt3v[pl.ds(c * chunk + off, L)] = v
        return 0

    lax.fori_loop(0, chunk // (NB * L), block, 0)
    Ppad = chunk * NW
    outs = [
        pltpu.async_copy(cenv, cen_hbm.at[pl.ds(base, chunk)], sem),
        pltpu.async_copy(o0v, out0_hbm.at[pl.ds(base, chunk)], sem),
        pltpu.async_copy(o1v, out1_hbm.at[pl.ds(base, chunk)], sem),
        pltpu.async_copy(o2v, out2_hbm.at[pl.ds(base, chunk)], sem),
    ]
    for c in range(4):
        outs.append(pltpu.async_copy(
            btv.at[pl.ds(c * chunk, chunk)],
            bt_hbm.at[pl.ds(c * Ppad + base, chunk)], sem))
    for c in range(9):
        outs.append(pltpu.async_copy(
            bt3v.at[pl.ds(c * chunk, chunk)],
            bt3_hbm.at[pl.ds(c * Ppad + base, chunk)], sem))
    for c in outs:
        c.wait()


def kernel(gt_bboxes, gt_labels, gt_bboxes_3d, gt_labels_3d, centers2d,
           depths, attr_labels, points, regress_ranges, stride_pt):
    P = points.shape[0]
    blk = NB * L
    chunk = ((P + NW - 1) // NW + blk - 1) // blk * blk
    Ppad = chunk * NW

    pad = Ppad - P
    in2d = jnp.concatenate([
        jnp.stack([points[:, 0], points[:, 1], regress_ranges[:, 0],
                   regress_ranges[:, 1], stride_pt]),
        jnp.ones((5, pad), jnp.float32),
    ], axis=1)

    tab = jnp.concatenate([
        centers2d.T,
        gt_bboxes.T,
        depths[None, :],
        gt_bboxes_3d[:, 3:9].T,
        gt_bboxes_3d[:, 0][None, :],
        gt_bboxes_3d[:, 2][None, :],
        lax.bitcast_convert_type(
            jnp.stack([gt_labels, gt_labels_3d, attr_labels]).astype(jnp.int32),
            jnp.float32),
    ]).astype(jnp.float32).reshape(-1)
    tft = jnp.concatenate([
        jnp.concatenate([centers2d, gt_bboxes,
                         jnp.zeros((64, L - 6), jnp.float32)], axis=1),
        jnp.full((2, L), 1e12, jnp.float32),
    ], axis=0).astype(jnp.float32).reshape(-1)

    mesh = plsc.VectorSubcoreMesh(core_axis_name="c", subcore_axis_name="s")
    kfn = functools.partial(
        pl.kernel,
        out_type=[jax.ShapeDtypeStruct((4 * Ppad,), jnp.float32),
                  jax.ShapeDtypeStruct((9 * Ppad,), jnp.float32),
                  jax.ShapeDtypeStruct((Ppad,), jnp.float32),
                  jax.ShapeDtypeStruct((Ppad,), jnp.int32),
                  jax.ShapeDtypeStruct((Ppad,), jnp.int32),
                  jax.ShapeDtypeStruct((Ppad,), jnp.int32)],
        mesh=mesh,
        compiler_params=pltpu.CompilerParams(needs_layout_passes=False),
        scratch_types=[pltpu.VMEM((5, chunk), jnp.float32),
                       pltpu.VMEM((18 * 64,), jnp.float32),
                       pltpu.VMEM((66 * L,), jnp.float32),
                       pltpu.VMEM((80,), jnp.int32),
                       pltpu.VMEM((chunk * 4,), jnp.float32),
                       pltpu.VMEM((chunk * 9,), jnp.float32),
                       pltpu.VMEM((chunk,), jnp.float32),
                       pltpu.VMEM((chunk,), jnp.int32),
                       pltpu.VMEM((chunk,), jnp.int32),
                       pltpu.VMEM((chunk,), jnp.int32),
                       pltpu.SemaphoreType.DMA],
    )(functools.partial(_body, chunk=chunk))
    btf, bt3f, cenf, o0, o1, o2 = kfn(in2d, tab, tft)

    labels = o0[:P]
    labels_3d = o1[:P]
    attrs = o2[:P]
    b2 = btf.reshape(4, Ppad)
    b3 = bt3f.reshape(9, Ppad)
    bt = jnp.stack([b2[c, :P] for c in range(4)], axis=-1)
    bt3d = jnp.stack([b3[c, :P] for c in range(9)], axis=-1)
    cent = cenf[:P]
    return (labels, bt, labels_3d, bt3d, cent, attrs)
